# baseline (speedup 1.0000x reference)
import dataclasses
import functools

import jax
import jax.numpy as jnp
from jax import lax
from jax.experimental import pallas as pl
from jax.experimental.pallas import tpu as pltpu
from jax.experimental.pallas import tpu_sc as plsc

_N = 10000
_E = 320000
_E2 = _E // 2
_D = 128
_H = 128

_NW = 32
_SEG_CHUNK = 200
_NCHUNK = _E2 // _SEG_CHUNK
_CHUNKS_PER_W = _NCHUNK // _NW
_NACC = 208
_BT = 256
_BI = 128
_SUBS = 64
_NSUB = _E // _SUBS


def _extract(vec, lane, idx):
    return jnp.sum(jnp.where(lane == idx, vec, 0))


def _segsum_body(he_hbm, eid_hbm, sub_hbm, pooled_hbm,
                 accs, rows, ids, lids, subv, winv, zvmem):
    cid = lax.axis_index("c")
    sid = lax.axis_index("s")
    w = cid * 16 + sid
    lane = lax.iota(jnp.int32, 16)
    acc = accs.at[sid]

    @pl.loop(0, _NACC)
    def _(r):
        for k in range(_D // 16):
            zvmem[r, pl.ds(k * 16, 16)] = jnp.zeros((16,), jnp.float32)

    pltpu.sync_copy(sub_hbm, subv)
    c0 = w * _CHUNKS_PER_W
    jstars = []
    for g in range(2):
        bval = (c0 + g * 16 + lane) * _SEG_CHUNK
        lo_v = jnp.zeros((16,), jnp.int32)
        hi_v = jnp.full((16,), _NSUB, jnp.int32)
        for _ in range(13):
            mid = jnp.minimum(lax.shift_right_logical(lo_v + hi_v, 1),
                              jnp.int32(_NSUB - 1))
            v = plsc.load_gather(subv, [mid])
            pred = v < bval
            lo_v = jnp.where(pred, mid + 1, lo_v)
            hi_v = jnp.where(pred, hi_v, mid)
        jstars.append(lo_v)

    def exact_bound(b):
        jstar = _extract(jstars[b // 16], lane, jnp.int32(b % 16))
        wb = jnp.maximum((jstar - 1) * _SUBS, 0)
        wb = pl.multiple_of(wb, 8)
        pltpu.sync_copy(eid_hbm.at[pl.ds(wb, _SUBS)], winv)
        bval = (c0 + b) * _SEG_CHUNK
        cnt = jnp.zeros((16,), jnp.int32)
        for k in range(_SUBS // 16):
            vv = winv[pl.ds(k * 16, 16)]
            cnt = cnt + jnp.where(vv < bval, 1, 0)
        return wb + jnp.sum(cnt)

    prev_hi = exact_bound(0)
    for t in range(_CHUNKS_PER_W):
        c = c0 + t
        seg_base = pl.multiple_of(c * _SEG_CHUNK, 8)
        lo = prev_hi
        hi = exact_bound(t + 1)
        prev_hi = hi
        pltpu.sync_copy(zvmem, acc)
        lo_a = lax.bitwise_and(lo, jnp.int32(-8))
        nbat = lax.div(hi - lo_a + jnp.int32(_BT - 1), jnp.int32(_BT))

        @pl.loop(0, nbat)
        def _(k):
            pos = lo_a + k * _BT
            posc = jnp.minimum(pos, jnp.int32(_E - _BT))
            posc = pl.multiple_of(posc, 8)
            pltpu.sync_copy(he_hbm.at[pl.ds(posc, _BT)], rows)
            pltpu.sync_copy(eid_hbm.at[pl.ds(posc, _BT)], ids)
            vlo = jnp.maximum(pos, lo)
            for j in range(_BT // 16):
                v = ids[pl.ds(j * 16, 16)]
                ridx = posc + j * 16 + lane
                valid = (ridx >= vlo) & (ridx < hi)
                lid = jnp.where(valid, v - seg_base, jnp.int32(_NACC - 1))
                lid = jnp.minimum(jnp.maximum(lid, 0), jnp.int32(_NACC - 1))
                r, col = (j * 16) // _BI, (j * 16) % _BI
                lids[r, pl.ds(col, 16)] = lid
            for r in range(_BT // _BI):
                pltpu.sync_copy(rows.at[pl.ds(r * _BI, _BI)],
                                acc.at[lids.at[r]], add=True)

        pltpu.sync_copy(acc.at[pl.ds(0, _SEG_CHUNK)],
                        pooled_hbm.at[pl.ds(seg_base, _SEG_CHUNK)])


def _sc_segment_sum(h_e, edge_id, sub):
    mesh = plsc.VectorSubcoreMesh(core_axis_name="c", subcore_axis_name="s")
    cp = pltpu.CompilerParams()
    if "needs_layout_passes" in pltpu.CompilerParams.__dataclass_fields__:
        cp = dataclasses.replace(cp, needs_layout_passes=False)
    f = pl.kernel(
        _segsum_body,
        out_type=jax.ShapeDtypeStruct((_E2, _D), jnp.float32),
        mesh=mesh,
        scratch_types=[
            pltpu.VMEM_SHARED((16, _NACC, _D), jnp.float32),
            pltpu.VMEM((_BT, _D), jnp.float32),
            pltpu.VMEM((_BT,), jnp.int32),
            pltpu.VMEM((_BT // _BI, _BI), jnp.int32),
            pltpu.VMEM((_NSUB,), jnp.int32),
            pltpu.VMEM((_SUBS,), jnp.int32),
            pltpu.VMEM((_NACC, _D), jnp.float32),
        ],
        compiler_params=cp,
    )
    return f(h_e, edge_id, sub)


def _mlp_block(x_ref, w1_ref, b1_ref, w2_ref, b2_ref, o_ref, *, bf16):
    x, w1, w2 = x_ref[...], w1_ref[...], w2_ref[...]
    if bf16:
        x, w1, w2 = (v.astype(jnp.bfloat16) for v in (x, w1, w2))
    h = jnp.dot(x, w1, preferred_element_type=jnp.float32)
    h = jnp.maximum(h + b1_ref[...], 0.0)
    if bf16:
        h = h.astype(jnp.bfloat16)
    o_ref[...] = (
        jnp.dot(h, w2, preferred_element_type=jnp.float32)
        + b2_ref[...]
    )


def _tc_mlp(x, w1, b1, w2, b2, blk, bf16=False):
    m, d = x.shape
    dout = w2.shape[1]
    grid = m // blk
    return pl.pallas_call(
        functools.partial(_mlp_block, bf16=bf16),
        grid=(grid,),
        in_specs=[
            pl.BlockSpec((blk, d), lambda i: (i, 0)),
            pl.BlockSpec(w1.shape, lambda i: (0, 0)),
            pl.BlockSpec((1, w1.shape[1]), lambda i: (0, 0)),
            pl.BlockSpec(w2.shape, lambda i: (0, 0)),
            pl.BlockSpec((1, dout), lambda i: (0, 0)),
        ],
        out_specs=pl.BlockSpec((blk, dout), lambda i: (i, 0)),
        out_shape=jax.ShapeDtypeStruct((m, dout), jnp.float32),
    )(x, w1, b1.reshape(1, -1), w2, b2.reshape(1, -1))


def kernel(h_v, edge_index, h_e, edge_id, Wn1, bn1, Wn2, bn2,
           We1, be1, We2, be2):
    sub = edge_id[::_SUBS]
    pooled = _sc_segment_sum(h_e, edge_id, sub)
    y_v = _tc_mlp(h_v, Wn1, bn1, Wn2, bn2, blk=1000)
    tension = _tc_mlp(pooled, We1, be1, We2, be2, blk=2000, bf16=True)
    return (y_v, tension.reshape(_E2))


# speedup vs baseline: 1.2908x; 1.2908x over previous
import dataclasses
import functools

import jax
import jax.numpy as jnp
from jax import lax
from jax.experimental import pallas as pl
from jax.experimental.pallas import tpu as pltpu
from jax.experimental.pallas import tpu_sc as plsc

_N = 10000
_E = 320000
_E2 = _E // 2
_D = 128
_H = 128

_NW = 32
_SEG_CHUNK = 200
_NCHUNK = _E2 // _SEG_CHUNK
_CHUNKS_PER_W = _NCHUNK // _NW
_NACC = 208
_BT = 256
_BI = 128
_SUBS = 64
_NSUB = _E // _SUBS


def _extract(vec, lane, idx):
    return jnp.sum(jnp.where(lane == idx, vec, 0))


def _segsum_body(he_hbm, eid_hbm, sub_hbm, pooled_hbm,
                 accs, rows, ids, lids, subv, winv, zvmem):
    cid = lax.axis_index("c")
    sid = lax.axis_index("s")
    w = cid * 16 + sid
    lane = lax.iota(jnp.int32, 16)
    acc = accs.at[sid]

    @pl.loop(0, _NACC)
    def _(r):
        for k in range(_D // 16):
            zvmem[r, pl.ds(k * 16, 16)] = jnp.zeros((16,), jnp.float32)

    pltpu.sync_copy(sub_hbm, subv)
    c0 = w * _CHUNKS_PER_W
    jstars = []
    for g in range(2):
        bval = (c0 + g * 16 + lane) * _SEG_CHUNK
        lo_v = jnp.zeros((16,), jnp.int32)
        hi_v = jnp.full((16,), _NSUB, jnp.int32)
        for _ in range(13):
            mid = jnp.minimum(lax.shift_right_logical(lo_v + hi_v, 1),
                              jnp.int32(_NSUB - 1))
            v = plsc.load_gather(subv, [mid])
            pred = v < bval
            lo_v = jnp.where(pred, mid + 1, lo_v)
            hi_v = jnp.where(pred, hi_v, mid)
        jstars.append(lo_v)

    def exact_bound(b):
        jstar = _extract(jstars[b // 16], lane, jnp.int32(b % 16))
        wb = jnp.maximum((jstar - 1) * _SUBS, 0)
        wb = pl.multiple_of(wb, 8)
        pltpu.sync_copy(eid_hbm.at[pl.ds(wb, _SUBS)], winv)
        bval = (c0 + b) * _SEG_CHUNK
        cnt = jnp.zeros((16,), jnp.int32)
        for k in range(_SUBS // 16):
            vv = winv[pl.ds(k * 16, 16)]
            cnt = cnt + jnp.where(vv < bval, 1, 0)
        return wb + jnp.sum(cnt)

    prev_hi = exact_bound(0)
    for t in range(_CHUNKS_PER_W):
        c = c0 + t
        seg_base = pl.multiple_of(c * _SEG_CHUNK, 8)
        lo = prev_hi
        hi = exact_bound(t + 1)
        prev_hi = hi
        pltpu.sync_copy(zvmem, acc)
        lo_a = lax.bitwise_and(lo, jnp.int32(-8))
        nbat = lax.div(hi - lo_a + jnp.int32(_BT - 1), jnp.int32(_BT))

        @pl.loop(0, nbat)
        def _(k):
            pos = lo_a + k * _BT
            posc = jnp.minimum(pos, jnp.int32(_E - _BT))
            posc = pl.multiple_of(posc, 8)
            pltpu.sync_copy(he_hbm.at[pl.ds(posc, _BT)], rows)
            pltpu.sync_copy(eid_hbm.at[pl.ds(posc, _BT)], ids)
            vlo = jnp.maximum(pos, lo)
            for j in range(_BT // 16):
                v = ids[pl.ds(j * 16, 16)]
                ridx = posc + j * 16 + lane
                valid = (ridx >= vlo) & (ridx < hi)
                lid = jnp.where(valid, v - seg_base, jnp.int32(_NACC - 1))
                lid = jnp.minimum(jnp.maximum(lid, 0), jnp.int32(_NACC - 1))
                r, col = (j * 16) // _BI, (j * 16) % _BI
                lids[r, pl.ds(col, 16)] = lid
            for r in range(_BT // _BI):
                pltpu.sync_copy(rows.at[pl.ds(r * _BI, _BI)],
                                acc.at[lids.at[r]], add=True)

        pltpu.sync_copy(acc.at[pl.ds(0, _SEG_CHUNK)],
                        pooled_hbm.at[pl.ds(seg_base, _SEG_CHUNK)])


def _sc_segment_sum(h_e, edge_id, sub):
    mesh = plsc.VectorSubcoreMesh(core_axis_name="c", subcore_axis_name="s")
    cp = pltpu.CompilerParams()
    if "needs_layout_passes" in pltpu.CompilerParams.__dataclass_fields__:
        cp = dataclasses.replace(cp, needs_layout_passes=False)
    f = pl.kernel(
        _segsum_body,
        out_type=jax.ShapeDtypeStruct((_E2, _D), jnp.float32),
        mesh=mesh,
        scratch_types=[
            pltpu.VMEM_SHARED((16, _NACC, _D), jnp.float32),
            pltpu.VMEM((_BT, _D), jnp.float32),
            pltpu.VMEM((_BT,), jnp.int32),
            pltpu.VMEM((_BT // _BI, _BI), jnp.int32),
            pltpu.VMEM((_NSUB,), jnp.int32),
            pltpu.VMEM((_SUBS,), jnp.int32),
            pltpu.VMEM((_NACC, _D), jnp.float32),
        ],
        compiler_params=cp,
    )
    return f(h_e, edge_id, sub)


def _mlp_block(x_ref, w1_ref, b1_ref, w2_ref, b2_ref, o_ref, *, bf16):
    x, w1, w2 = x_ref[...], w1_ref[...], w2_ref[...]
    if bf16:
        x, w1, w2 = (v.astype(jnp.bfloat16) for v in (x, w1, w2))
    h = jnp.dot(x, w1, preferred_element_type=jnp.float32)
    h = jnp.maximum(h + b1_ref[...], 0.0)
    if bf16:
        h = h.astype(jnp.bfloat16)
    o_ref[...] = (
        jnp.dot(h, w2, preferred_element_type=jnp.float32)
        + b2_ref[...]
    )


def _tc_mlp(x, w1, b1, w2, b2, blk, bf16=False):
    m, d = x.shape
    dout = w2.shape[1]
    grid = m // blk
    return pl.pallas_call(
        functools.partial(_mlp_block, bf16=bf16),
        grid=(grid,),
        in_specs=[
            pl.BlockSpec((blk, d), lambda i: (i, 0)),
            pl.BlockSpec(w1.shape, lambda i: (0, 0)),
            pl.BlockSpec((1, w1.shape[1]), lambda i: (0, 0)),
            pl.BlockSpec(w2.shape, lambda i: (0, 0)),
            pl.BlockSpec((1, dout), lambda i: (0, 0)),
        ],
        out_specs=pl.BlockSpec((blk, dout), lambda i: (i, 0)),
        out_shape=jax.ShapeDtypeStruct((m, dout), jnp.float32),
    )(x, w1, b1.reshape(1, -1), w2, b2.reshape(1, -1))


def _edge_mlp_block(x_ref, w1_ref, b1_ref, w2_ref, b2_ref, o_ref):
    x = x_ref[...].astype(jnp.bfloat16)
    w1 = w1_ref[...].astype(jnp.bfloat16)
    h = jnp.dot(x, w1, preferred_element_type=jnp.float32)
    h = jnp.maximum(h + b1_ref[...], 0.0).astype(jnp.bfloat16)
    w2 = w2_ref[...].astype(jnp.bfloat16)
    row = jax.lax.dot_general(w2, h, (((0,), (1,)), ((), ())),
                              preferred_element_type=jnp.float32)
    o_ref[...] = row.reshape(_EBLK) + b2_ref[0, 0]


_EBLK = 16384


def _tc_edge_mlp(x, w1, b1, w2, b2):
    grid = 10
    out = pl.pallas_call(
        _edge_mlp_block,
        grid=(grid,),
        in_specs=[
            pl.BlockSpec((_EBLK, _D), lambda i: (i, 0)),
            pl.BlockSpec((_D, _H), lambda i: (0, 0)),
            pl.BlockSpec((1, _H), lambda i: (0, 0)),
            pl.BlockSpec((_H, 1), lambda i: (0, 0)),
            pl.BlockSpec((1, 1), lambda i: (0, 0)),
        ],
        out_specs=pl.BlockSpec((_EBLK,), lambda i: (i,)),
        out_shape=jax.ShapeDtypeStruct((grid * _EBLK,), jnp.float32),
    )(x, w1, b1.reshape(1, -1), w2, b2.reshape(1, -1))
    return out[:_E2]


def kernel(h_v, edge_index, h_e, edge_id, Wn1, bn1, Wn2, bn2,
           We1, be1, We2, be2):
    sub = edge_id[::_SUBS]
    pooled = _sc_segment_sum(h_e, edge_id, sub)
    y_v = _tc_mlp(h_v, Wn1, bn1, Wn2, bn2, blk=1000)
    tension = _tc_edge_mlp(pooled, We1, be1, We2, be2)
    return (y_v, tension)


# speedup vs baseline: 1.8168x; 1.4075x over previous
import dataclasses
import functools

import jax
import jax.numpy as jnp
from jax import lax
from jax.experimental import pallas as pl
from jax.experimental.pallas import tpu as pltpu
from jax.experimental.pallas import tpu_sc as plsc

_N = 10000
_E = 320000
_E2 = _E // 2
_D = 128
_H = 128

_NW = 32
_SEG_CHUNK = 200
_NCHUNK = _E2 // _SEG_CHUNK
_CHUNKS_PER_W = _NCHUNK // _NW
_NACC = 208
_BT = 256
_BI = 128
_SUBS = 64
_NSUB = _E // _SUBS


def _extract(vec, lane, idx):
    return jnp.sum(jnp.where(lane == idx, vec, 0))


def _segsum_body(he_hbm, eid_hbm, sub_hbm, pooled_hbm,
                 accs, rows, ids0, ids1, lids, subv, winv, zvmem,
                 sem_r0, sem_r1, sem_i0, sem_i1, sem_o0, sem_o1):
    cid = lax.axis_index("c")
    sid = lax.axis_index("s")
    w = cid * 16 + sid
    lane = lax.iota(jnp.int32, 16)
    sem_r = (sem_r0, sem_r1)
    sem_i = (sem_i0, sem_i1)
    sem_o = (sem_o0, sem_o1)
    ids = (ids0, ids1)

    @pl.loop(0, _NACC)
    def _(r):
        for k in range(_D // 16):
            zvmem[r, pl.ds(k * 16, 16)] = jnp.zeros((16,), jnp.float32)

    pltpu.sync_copy(sub_hbm, subv)
    c0 = w * _CHUNKS_PER_W
    jstars = []
    for g in range(2):
        bval = (c0 + g * 16 + lane) * _SEG_CHUNK
        lo_v = jnp.zeros((16,), jnp.int32)
        hi_v = jnp.full((16,), _NSUB, jnp.int32)
        for _ in range(13):
            mid = jnp.minimum(lax.shift_right_logical(lo_v + hi_v, 1),
                              jnp.int32(_NSUB - 1))
            v = plsc.load_gather(subv, [mid])
            pred = v < bval
            lo_v = jnp.where(pred, mid + 1, lo_v)
            hi_v = jnp.where(pred, hi_v, mid)
        jstars.append(lo_v)

    def exact_bound(b):
        jstar = _extract(jstars[b // 16], lane, jnp.int32(b % 16))
        wb = jnp.maximum((jstar - 1) * _SUBS, 0)
        wb = pl.multiple_of(wb, 8)
        pltpu.sync_copy(eid_hbm.at[pl.ds(wb, _SUBS)], winv)
        bval = (c0 + b) * _SEG_CHUNK
        cnt = jnp.zeros((16,), jnp.int32)
        for k in range(_SUBS // 16):
            vv = winv[pl.ds(k * 16, 16)]
            cnt = cnt + jnp.where(vv < bval, 1, 0)
        return wb + jnp.sum(cnt)

    def batch_pos(lo_a, k):
        pos = lo_a + k * _BT
        posc = jnp.minimum(pos, jnp.int32(_E - _BT))
        return pos, pl.multiple_of(posc, 8)

    def issue_gather(lo_a, k, b):
        _, posc = batch_pos(lo_a, k)
        pltpu.async_copy(he_hbm.at[pl.ds(posc, _BT)], rows.at[b], sem_r[b])
        pltpu.async_copy(eid_hbm.at[pl.ds(posc, _BT)], ids[b], sem_i[b])

    bh = [exact_bound(0), exact_bound(1)]
    for t in range(_CHUNKS_PER_W):
        s = t % 2
        acc = accs.at[sid]
        c = c0 + t
        seg_base = pl.multiple_of(c * _SEG_CHUNK, 8)
        lo, hi = bh[t], bh[t + 1]
        lo_a = lax.bitwise_and(lo, jnp.int32(-8))
        nbat = lax.div(hi - lo_a + jnp.int32(_BT - 1), jnp.int32(_BT))

        @pl.when(nbat > 0)
        def _():
            issue_gather(lo_a, 0, 0)
        if t + 2 <= _CHUNKS_PER_W:
            bh.append(exact_bound(t + 2))
        if t >= 1:
            pltpu.make_async_copy(
                acc.at[pl.ds(0, _SEG_CHUNK)],
                pooled_hbm.at[pl.ds(seg_base, _SEG_CHUNK)],
                sem_o[(t - 1) % 2]).wait()
        pltpu.sync_copy(zvmem, acc)

        @pl.loop(0, lax.bitwise_and(nbat + 1, jnp.int32(-2)), step=2)
        def _(k):
            for b in range(2):
                kk = k + b

                @pl.when(kk < nbat)
                def _():
                    pltpu.make_async_copy(he_hbm.at[pl.ds(0, _BT)],
                                          rows.at[b], sem_r[b]).wait()
                    pltpu.make_async_copy(eid_hbm.at[pl.ds(0, _BT)],
                                          ids[b], sem_i[b]).wait()

                    @pl.when(kk + 1 < nbat)
                    def _():
                        issue_gather(lo_a, kk + 1, 1 - b)

                    pos, posc = batch_pos(lo_a, kk)
                    vlo = jnp.maximum(pos, lo)
                    for j in range(_BT // 16):
                        v = ids[b][pl.ds(j * 16, 16)]
                        ridx = posc + j * 16 + lane
                        valid = (ridx >= vlo) & (ridx < hi)
                        lid = jnp.where(valid, v - seg_base,
                                        jnp.int32(_NACC - 1))
                        lid = jnp.minimum(jnp.maximum(lid, 0),
                                          jnp.int32(_NACC - 1))
                        r, col = (j * 16) // _BI, (j * 16) % _BI
                        lids[r, pl.ds(col, 16)] = lid
                    for r in range(_BT // _BI):
                        pltpu.sync_copy(
                            rows.at[b].at[pl.ds(r * _BI, _BI)],
                            acc.at[lids.at[r]], add=True)

        pltpu.async_copy(acc.at[pl.ds(0, _SEG_CHUNK)],
                         pooled_hbm.at[pl.ds(seg_base, _SEG_CHUNK)],
                         sem_o[s])

    t = _CHUNKS_PER_W - 1
    seg_base = pl.multiple_of((c0 + t) * _SEG_CHUNK, 8)
    pltpu.make_async_copy(
        accs.at[sid].at[pl.ds(0, _SEG_CHUNK)],
        pooled_hbm.at[pl.ds(seg_base, _SEG_CHUNK)],
        sem_o[t % 2]).wait()


def _sc_segment_sum(h_e, edge_id, sub):
    mesh = plsc.VectorSubcoreMesh(core_axis_name="c", subcore_axis_name="s")
    cp = pltpu.CompilerParams()
    if "needs_layout_passes" in pltpu.CompilerParams.__dataclass_fields__:
        cp = dataclasses.replace(cp, needs_layout_passes=False)
    f = pl.kernel(
        _segsum_body,
        out_type=jax.ShapeDtypeStruct((_E2, _D), jnp.float32),
        mesh=mesh,
        scratch_types=[
            pltpu.VMEM_SHARED((16, _NACC, _D), jnp.float32),
            pltpu.VMEM((2, _BT, _D), jnp.float32),
            pltpu.VMEM((_BT,), jnp.int32),
            pltpu.VMEM((_BT,), jnp.int32),
            pltpu.VMEM((_BT // _BI, _BI), jnp.int32),
            pltpu.VMEM((_NSUB,), jnp.int32),
            pltpu.VMEM((_SUBS,), jnp.int32),
            pltpu.VMEM((_NACC, _D), jnp.float32),
            pltpu.SemaphoreType.DMA,
            pltpu.SemaphoreType.DMA,
            pltpu.SemaphoreType.DMA,
            pltpu.SemaphoreType.DMA,
            pltpu.SemaphoreType.DMA,
            pltpu.SemaphoreType.DMA,
        ],
        compiler_params=cp,
    )
    return f(h_e, edge_id, sub)


def _mlp_block(x_ref, w1_ref, b1_ref, w2_ref, b2_ref, o_ref, *, bf16):
    x, w1, w2 = x_ref[...], w1_ref[...], w2_ref[...]
    if bf16:
        x, w1, w2 = (v.astype(jnp.bfloat16) for v in (x, w1, w2))
    h = jnp.dot(x, w1, preferred_element_type=jnp.float32)
    h = jnp.maximum(h + b1_ref[...], 0.0)
    if bf16:
        h = h.astype(jnp.bfloat16)
    o_ref[...] = (
        jnp.dot(h, w2, preferred_element_type=jnp.float32)
        + b2_ref[...]
    )


def _tc_mlp(x, w1, b1, w2, b2, blk, bf16=False):
    m, d = x.shape
    dout = w2.shape[1]
    grid = m // blk
    return pl.pallas_call(
        functools.partial(_mlp_block, bf16=bf16),
        grid=(grid,),
        in_specs=[
            pl.BlockSpec((blk, d), lambda i: (i, 0)),
            pl.BlockSpec(w1.shape, lambda i: (0, 0)),
            pl.BlockSpec((1, w1.shape[1]), lambda i: (0, 0)),
            pl.BlockSpec(w2.shape, lambda i: (0, 0)),
            pl.BlockSpec((1, dout), lambda i: (0, 0)),
        ],
        out_specs=pl.BlockSpec((blk, dout), lambda i: (i, 0)),
        out_shape=jax.ShapeDtypeStruct((m, dout), jnp.float32),
    )(x, w1, b1.reshape(1, -1), w2, b2.reshape(1, -1))


def _edge_mlp_block(x_ref, w1_ref, b1_ref, w2_ref, b2_ref, o_ref):
    x = x_ref[...].astype(jnp.bfloat16)
    w1 = w1_ref[...].astype(jnp.bfloat16)
    h = jnp.dot(x, w1, preferred_element_type=jnp.float32)
    h = jnp.maximum(h + b1_ref[...], 0.0).astype(jnp.bfloat16)
    w2 = w2_ref[...].astype(jnp.bfloat16)
    row = jax.lax.dot_general(w2, h, (((0,), (1,)), ((), ())),
                              preferred_element_type=jnp.float32)
    o_ref[...] = row.reshape(_EBLK) + b2_ref[0, 0]


_EBLK = 16384


def _tc_edge_mlp(x, w1, b1, w2, b2):
    grid = 10
    out = pl.pallas_call(
        _edge_mlp_block,
        grid=(grid,),
        in_specs=[
            pl.BlockSpec((_EBLK, _D), lambda i: (i, 0)),
            pl.BlockSpec((_D, _H), lambda i: (0, 0)),
            pl.BlockSpec((1, _H), lambda i: (0, 0)),
            pl.BlockSpec((_H, 1), lambda i: (0, 0)),
            pl.BlockSpec((1, 1), lambda i: (0, 0)),
        ],
        out_specs=pl.BlockSpec((_EBLK,), lambda i: (i,)),
        out_shape=jax.ShapeDtypeStruct((grid * _EBLK,), jnp.float32),
    )(x, w1, b1.reshape(1, -1), w2, b2.reshape(1, -1))
    return out[:_E2]


def kernel(h_v, edge_index, h_e, edge_id, Wn1, bn1, Wn2, bn2,
           We1, be1, We2, be2):
    sub = edge_id[::_SUBS]
    pooled = _sc_segment_sum(h_e, edge_id, sub)
    y_v = _tc_mlp(h_v, Wn1, bn1, Wn2, bn2, blk=1000)
    tension = _tc_edge_mlp(pooled, We1, be1, We2, be2)
    return (y_v, tension)
